# baseline (speedup 1.0000x reference)
import functools

import jax
import jax.numpy as jnp
from jax import lax
from jax.experimental import pallas as pl
from jax.experimental.pallas import tpu as pltpu
from jax.experimental.pallas import tpu_sc as plsc

EPS = 1e-5
EMBED = 64
LANES = 16
ROWS_PER_CHUNK = 4
NBUF = 2


def _full(v):
    return jnp.full((LANES,), v, dtype=jnp.int32)


def _rsqrt(x):
    i = plsc.bitcast(x, jnp.int32)
    i = jnp.int32(0x5F3759DF) - (i >> 1)
    y = plsc.bitcast(i, jnp.float32)
    for _ in range(2):
        y = y * (1.5 - 0.5 * x * y * y)
    return y


def _make_sc_kernel(nb, nt):
    info = plsc.get_sparse_core_info()
    nc, ns = info.num_cores, info.num_subcores
    nw = nc * ns
    b_per_w = nb // nw
    n_chunks = b_per_w // ROWS_PER_CHUNK
    chunk = ROWS_PER_CHUNK * nt
    groups = chunk // LANES
    half = nt // 2

    mesh = plsc.VectorSubcoreMesh(core_axis_name="c", subcore_axis_name="s")

    @functools.partial(
        pl.kernel,
        mesh=mesh,
        compiler_params=pltpu.CompilerParams(
            needs_layout_passes=False, use_tc_tiling_on_sc=False
        ),
        out_type=jax.ShapeDtypeStruct((nb, nt, EMBED), jnp.float32),
        scratch_types=[
            pltpu.VMEM((NBUF, chunk), jnp.int32),
            pltpu.VMEM((NBUF, chunk, EMBED), jnp.float32),
            pltpu.VMEM((EMBED,), jnp.float32),
            pltpu.VMEM((EMBED,), jnp.float32),
            pltpu.SemaphoreType.DMA,
            pltpu.SemaphoreType.DMA,
        ],
    )
    def sc_kernel(x_hbm, table_hbm, w_hbm, b_hbm, out_hbm,
                  idx_v, rows_v, w_v, b_v, sem_g, sem_o):
        wid = lax.axis_index("s") * nc + lax.axis_index("c")
        w0 = wid * b_per_w
        pltpu.sync_copy(w_hbm, w_v)
        pltpu.sync_copy(b_hbm, b_v)
        wq = [w_v[pl.ds(q * LANES, LANES)] for q in range(4)]
        bq = [b_v[pl.ds(q * LANES, LANES)] for q in range(4)]

        def load_idx(c, b):
            base = pl.multiple_of((w0 + c * ROWS_PER_CHUNK) * nt, 8)
            pltpu.sync_copy(x_hbm.at[pl.ds(base, chunk)], idx_v.at[b])

        _splits = [(0, 96), (96, nt - 96)]

        def g_pairs(b):
            out = []
            for r in range(ROWS_PER_CHUNK):
                for off, sz in _splits:
                    src = idx_v.at[b].at[pl.ds(r * nt + off, sz)]
                    dst = rows_v.at[b].at[pl.ds(r * nt + off, sz)]
                    out.append((src, dst))
            return out

        def fire_gathers(b):
            for src, dst in g_pairs(b):
                pltpu.async_copy(table_hbm.at[src], dst, sem_g)

        def wait_gathers(b):
            for src, dst in g_pairs(b):
                pltpu.make_async_copy(table_hbm.at[src], dst, sem_g).wait()

        def o_pairs(c, b):
            out = []
            for r in range(ROWS_PER_CHUNK):
                src = rows_v.at[b].at[pl.ds(r * nt, nt)]
                dst = out_hbm.at[w0 + c * ROWS_PER_CHUNK + r]
                out.append((src, dst))
            return out

        def fire_out(c, b):
            for src, dst in o_pairs(c, b):
                pltpu.async_copy(src, dst, sem_o)

        def wait_out(c, b):
            for src, dst in o_pairs(c, b):
                pltpu.make_async_copy(src, dst, sem_o).wait()

        def compute(b):
            rows = rows_v.at[b]
            lane = lax.iota(jnp.int32, 16)
            rot = [(lane + jj) & 15 for jj in range(LANES)]

            @plsc.parallel_loop(0, groups, unroll=2)
            def group_body(g):
                r0 = g * LANES
                rid = r0 + lane
                nacc = 8
                acc = [jnp.zeros((LANES,), jnp.float32) for _ in range(nacc)]
                acc2 = [jnp.zeros((LANES,), jnp.float32) for _ in range(nacc)]
                for j in range(EMBED):
                    q, jj = divmod(j, LANES)
                    col = plsc.load_gather(rows, [rid, rot[jj] | (q * LANES)])
                    k = j % nacc
                    acc[k] = acc[k] + col
                    acc2[k] = acc2[k] + col * col
                while len(acc) > 1:
                    acc = [a + b2 for a, b2 in zip(acc[::2], acc[1::2])]
                    acc2 = [a + b2 for a, b2 in zip(acc2[::2], acc2[1::2])]
                s, s2 = acc[0], acc2[0]
                mean = s * (1.0 / EMBED)
                var = s2 * (1.0 / EMBED) - mean * mean
                rstd = _rsqrt(var + EPS)
                for r in range(LANES):
                    mb = mean.at[_full(r)].get(mode="promise_in_bounds")
                    rb = rstd.at[_full(r)].get(mode="promise_in_bounds")
                    for q in range(4):
                        sl = pl.ds(q * LANES, LANES)
                        v = rows[r0 + r, sl]
                        rows[r0 + r, sl] = (v - mb) * rb * wq[q] + bq[q]

        load_idx(0, 0)
        fire_gathers(0)

        def chunk_body(c, _):
            b = lax.rem(c, NBUF)
            b1 = lax.rem(c + 1, NBUF)
            wait_gathers(b)

            @pl.when(c >= 1)
            def _drain():
                wait_out(c - 1, b1)

            @pl.when(c + 1 < n_chunks)
            def _prefetch():
                load_idx(c + 1, b1)
                fire_gathers(b1)

            compute(b)
            fire_out(c, b)
            return 0

        lax.fori_loop(0, n_chunks, chunk_body, 0)
        wait_out(n_chunks - 1, lax.rem(n_chunks - 1, NBUF))

    return sc_kernel


def kernel(x, table, ln_weight, ln_bias):
    nb, nt = x.shape
    sc = _make_sc_kernel(nb, nt)
    return sc(x.reshape(-1).astype(jnp.int32), table, ln_weight, ln_bias)


# speedup vs baseline: 1.2099x; 1.2099x over previous
import functools

import jax
import jax.numpy as jnp
from jax import lax
from jax.experimental import pallas as pl
from jax.experimental.pallas import tpu as pltpu
from jax.experimental.pallas import tpu_sc as plsc

EPS = 1e-5
EMBED = 64
LANES = 16
ROWS_PER_CHUNK = 2
NBUF = 2


def _full(v):
    return jnp.full((LANES,), v, dtype=jnp.int32)


def _rsqrt(x):
    i = plsc.bitcast(x, jnp.int32)
    i = jnp.int32(0x5F3759DF) - (i >> 1)
    y = plsc.bitcast(i, jnp.float32)
    for _ in range(2):
        y = y * (1.5 - 0.5 * x * y * y)
    return y


def _make_sc_kernel(nb, nt):
    info = plsc.get_sparse_core_info()
    nc, ns = info.num_cores, info.num_subcores
    nw = nc * ns
    b_per_w = nb // nw
    n_chunks = b_per_w // ROWS_PER_CHUNK
    chunk = ROWS_PER_CHUNK * nt
    groups = chunk // LANES
    half = nt // 2

    mesh = plsc.VectorSubcoreMesh(core_axis_name="c", subcore_axis_name="s")

    @functools.partial(
        pl.kernel,
        mesh=mesh,
        compiler_params=pltpu.CompilerParams(
            needs_layout_passes=False, use_tc_tiling_on_sc=False
        ),
        out_type=jax.ShapeDtypeStruct((nb, nt, EMBED), jnp.float32),
        scratch_types=[
            pltpu.VMEM((NBUF, chunk), jnp.int32),
            pltpu.VMEM((NBUF, chunk, EMBED), jnp.float32),
            pltpu.VMEM((EMBED,), jnp.float32),
            pltpu.VMEM((EMBED,), jnp.float32),
            pltpu.SemaphoreType.DMA,
            pltpu.SemaphoreType.DMA,
        ],
    )
    def sc_kernel(x_hbm, table_hbm, w_hbm, b_hbm, out_hbm,
                  idx_v, rows_v, w_v, b_v, sem_g, sem_o):
        wid = lax.axis_index("s") * nc + lax.axis_index("c")
        w0 = wid * b_per_w
        pltpu.sync_copy(w_hbm, w_v)
        pltpu.sync_copy(b_hbm, b_v)
        wq = [w_v[pl.ds(q * LANES, LANES)] for q in range(4)]
        bq = [b_v[pl.ds(q * LANES, LANES)] for q in range(4)]

        def load_idx(c, b):
            base = pl.multiple_of((w0 + c * ROWS_PER_CHUNK) * nt, 8)
            pltpu.sync_copy(x_hbm.at[pl.ds(base, chunk)], idx_v.at[b])

        _splits = [(0, 96), (96, nt - 96)]

        def g_pairs(b):
            out = []
            for r in range(ROWS_PER_CHUNK):
                for off, sz in _splits:
                    src = idx_v.at[b].at[pl.ds(r * nt + off, sz)]
                    dst = rows_v.at[b].at[pl.ds(r * nt + off, sz)]
                    out.append((src, dst))
            return out

        def fire_gathers(b):
            for src, dst in g_pairs(b):
                pltpu.async_copy(table_hbm.at[src], dst, sem_g)

        def wait_gathers(b):
            for src, dst in g_pairs(b):
                pltpu.make_async_copy(table_hbm.at[src], dst, sem_g).wait()

        def o_pairs(c, b):
            out = []
            for r in range(ROWS_PER_CHUNK):
                src = rows_v.at[b].at[pl.ds(r * nt, nt)]
                dst = out_hbm.at[w0 + c * ROWS_PER_CHUNK + r]
                out.append((src, dst))
            return out

        def fire_out(c, b):
            for src, dst in o_pairs(c, b):
                pltpu.async_copy(src, dst, sem_o)

        def wait_out(c, b):
            for src, dst in o_pairs(c, b):
                pltpu.make_async_copy(src, dst, sem_o).wait()

        def compute(b):
            rows = rows_v.at[b]
            lane = lax.iota(jnp.int32, 16)
            rot = [(lane + jj) & 15 for jj in range(LANES)]

            @plsc.parallel_loop(0, groups, unroll=1)
            def group_body(g):
                r0 = g * LANES
                rid = r0 + lane
                nacc = 8
                acc = [jnp.zeros((LANES,), jnp.float32) for _ in range(nacc)]
                acc2 = [jnp.zeros((LANES,), jnp.float32) for _ in range(nacc)]
                for j in range(EMBED):
                    q, jj = divmod(j, LANES)
                    col = plsc.load_gather(rows, [rid, rot[jj] | (q * LANES)])
                    k = j % nacc
                    acc[k] = acc[k] + col
                    acc2[k] = acc2[k] + col * col
                while len(acc) > 1:
                    acc = [a + b2 for a, b2 in zip(acc[::2], acc[1::2])]
                    acc2 = [a + b2 for a, b2 in zip(acc2[::2], acc2[1::2])]
                s, s2 = acc[0], acc2[0]
                mean = s * (1.0 / EMBED)
                var = s2 * (1.0 / EMBED) - mean * mean
                rstd = _rsqrt(var + EPS)
                for r in range(LANES):
                    mb = mean.at[_full(r)].get(mode="promise_in_bounds")
                    rb = rstd.at[_full(r)].get(mode="promise_in_bounds")
                    for q in range(4):
                        sl = pl.ds(q * LANES, LANES)
                        v = rows[r0 + r, sl]
                        rows[r0 + r, sl] = (v - mb) * rb * wq[q] + bq[q]

        load_idx(0, 0)
        fire_gathers(0)

        def chunk_body(c, _):
            b = lax.rem(c, NBUF)
            b1 = lax.rem(c + 1, NBUF)
            wait_gathers(b)

            @pl.when(c >= 1)
            def _drain():
                wait_out(c - 1, b1)

            @pl.when(c + 1 < n_chunks)
            def _prefetch():
                load_idx(c + 1, b1)
                fire_gathers(b1)

            compute(b)
            fire_out(c, b)
            return 0

        lax.fori_loop(0, n_chunks, chunk_body, 0)
        wait_out(n_chunks - 1, lax.rem(n_chunks - 1, NBUF))

    return sc_kernel


def kernel(x, table, ln_weight, ln_bias):
    nb, nt = x.shape
    sc = _make_sc_kernel(nb, nt)
    return sc(x.reshape(-1).astype(jnp.int32), table, ln_weight, ln_bias)
